# baseline (speedup 1.0000x reference)
import jax
import jax.numpy as jnp
from jax import lax
from jax.experimental import pallas as pl
from jax.experimental.pallas import tpu as pltpu
from jax.experimental.pallas import tpu_sc as plsc

N = 10000
D = 128
E = 320000
NC = 2
NS = 16
CH = 128
G = 1
KCH = 160
GRP = KCH // G
EPAD = NS * KCH * CH
NH = 5120
NHB = NH + 8
RPT = NH // NS
_COPIES = ((0, 128), (128, 128), (256, 64))

_mesh = plsc.VectorSubcoreMesh(core_axis_name="c", subcore_axis_name="s")


def _make_agg():
    scratch = [
        pltpu.VMEM((GRP, G * CH), jnp.int32),
        pltpu.VMEM((GRP, G * CH), jnp.int32),
        pltpu.VMEM((G * CH, D), jnp.float32),
        pltpu.VMEM_SHARED((NHB, D), jnp.float32),
        pltpu.SemaphoreType.DMA,
    ]

    def body(h_hbm, src_hbm, ldst_hbm, agg_out,
             src_v, ldst_v, rows_v, agg_sh, sem):
        c = lax.axis_index("c")
        s = lax.axis_index("s")

        def zf(i, carry):
            r = i // (D // 16)
            col = (i % (D // 16)) * 16
            rows_v[r, pl.ds(col, 16)] = jnp.zeros((16,), jnp.float32)
            return carry
        lax.fori_loop(0, 128 * (D // 16), zf, 0)

        for off, nr in _COPIES:
            pltpu.sync_copy(rows_v.at[pl.ds(0, nr)],
                            agg_sh.at[pl.ds(s * RPT + off, nr)])

        @pl.when(s == 0)
        def _zero_trash():
            pltpu.sync_copy(rows_v.at[pl.ds(0, 8)], agg_sh.at[pl.ds(NH, 8)])

        pltpu.sync_copy(src_hbm.at[s], src_v)
        pltpu.sync_copy(ldst_hbm.at[c, s], ldst_v)
        plsc.subcore_barrier()

        def step(g, carry):
            pltpu.async_copy(h_hbm.at[src_v.at[g]], rows_v,
                             sem).wait()
            pltpu.sync_copy(rows_v, agg_sh.at[ldst_v.at[g]],
                            add=True)
            return carry
        lax.fori_loop(0, GRP, step, 0)
        plsc.subcore_barrier()

        for off, nr in _COPIES:
            o = s * RPT + off
            pltpu.sync_copy(agg_sh.at[pl.ds(o, nr)],
                            agg_out.at[c, pl.ds(o, nr)])

    return pl.kernel(
        body,
        out_type=[jax.ShapeDtypeStruct((NC, NH, D), jnp.float32)],
        mesh=_mesh, scratch_types=scratch)


def _make_cnt():
    scratch = [
        pltpu.VMEM((GRP, G * CH), jnp.int32),
        pltpu.VMEM((G * CH, D), jnp.float32),
        pltpu.VMEM_SHARED((NHB, D), jnp.float32),
        pltpu.SemaphoreType.DMA,
    ]

    def body(ldst_hbm, cnt_out, ldst_v, ones_v, cnt_sh, sem):
        c = lax.axis_index("c")
        s = lax.axis_index("s")

        def zf(i, carry):
            r = i // (D // 16)
            col = (i % (D // 16)) * 16
            ones_v[r, pl.ds(col, 16)] = jnp.zeros((16,), jnp.float32)
            return carry
        lax.fori_loop(0, 128 * (D // 16), zf, 0)

        for off, nr in _COPIES:
            pltpu.sync_copy(ones_v.at[pl.ds(0, nr)],
                            cnt_sh.at[pl.ds(s * RPT + off, nr)])

        @pl.when(s == 0)
        def _zero_trash():
            pltpu.sync_copy(ones_v.at[pl.ds(0, 8)], cnt_sh.at[pl.ds(NH, 8)])

        def of(i, carry):
            r = i // (D // 16)
            col = (i % (D // 16)) * 16
            ones_v[r, pl.ds(col, 16)] = jnp.ones((16,), jnp.float32)
            return carry
        lax.fori_loop(0, G * CH * (D // 16), of, 0)

        pltpu.sync_copy(ldst_hbm.at[c, s], ldst_v)
        plsc.subcore_barrier()

        def step(g, carry):
            pltpu.async_copy(ones_v, cnt_sh.at[ldst_v.at[g]],
                             sem, add=True)

            @pl.when(g >= 4)
            def _drain_one():
                pltpu.make_async_copy(
                    ones_v, cnt_sh.at[ldst_v.at[g - 4]],
                    sem).wait()
            return carry
        lax.fori_loop(0, GRP, step, 0)
        for t in range(4):
            pltpu.make_async_copy(
                ones_v, cnt_sh.at[ldst_v.at[GRP - 4 + t]],
                sem).wait()
        plsc.subcore_barrier()

        for off, nr in _COPIES:
            o = s * RPT + off
            pltpu.sync_copy(cnt_sh.at[pl.ds(o, nr)],
                            cnt_out.at[c, pl.ds(o, nr)])

    return pl.kernel(
        body,
        out_type=[jax.ShapeDtypeStruct((NC, NH, D), jnp.float32)],
        mesh=_mesh, scratch_types=scratch)


_agg_k = _make_agg()
_cnt_k = _make_cnt()

R = 1000


def _mm_body(a_ref, cnt_ref, h_ref, wl_ref, wr_ref, b_ref, fl_ref, o_ref):
    inv = 1.0 / jnp.maximum(cnt_ref[:, 0:1], 1.0)
    mean = a_ref[...] * inv
    acc = jnp.dot(mean, wl_ref[...], preferred_element_type=jnp.float32)
    acc = acc + jnp.dot(h_ref[...], wr_ref[...],
                        preferred_element_type=jnp.float32)
    acc = acc + b_ref[...]
    o_ref[...] = jnp.where(fl_ref[...] > 0.0, jnp.maximum(acc, 0.0), acc)


_mm_k = pl.pallas_call(
    _mm_body,
    grid=(N // R,),
    in_specs=[
        pl.BlockSpec((R, D), lambda i: (i, 0)),
        pl.BlockSpec((R, D), lambda i: (i, 0)),
        pl.BlockSpec((R, D), lambda i: (i, 0)),
        pl.BlockSpec((D, D), lambda i: (0, 0)),
        pl.BlockSpec((D, D), lambda i: (0, 0)),
        pl.BlockSpec((1, D), lambda i: (0, 0)),
        pl.BlockSpec((1, D), lambda i: (0, 0)),
    ],
    out_specs=pl.BlockSpec((R, D), lambda i: (i, 0)),
    out_shape=jax.ShapeDtypeStruct((N, D), jnp.float32),
)


@jax.jit
def kernel(x, edge_index, W_l1, b_l1, W_r1, W_l2, b_l2, W_r2,
           W_l3, b_l3, W_r3):
    pad = EPAD - E
    src = jnp.concatenate([edge_index[0],
                           jnp.zeros((pad,), jnp.int32)]).reshape(NS, GRP, G * CH)
    dst = jnp.concatenate([edge_index[1], jnp.full((pad,), -1, jnp.int32)])
    base = jnp.arange(NC, dtype=jnp.int32)[:, None] * NH
    local = dst[None, :] - base
    ldst = jnp.where((local >= 0) & (local < NH), local, NH)
    ldst = ldst.reshape(NC, NS, GRP, G * CH)

    cnt, = _cnt_k(ldst)
    cnt = cnt.reshape(NC * NH, D)

    Wl = jnp.stack([W_l1, W_l2, W_l3])
    Wr = jnp.stack([W_r1, W_r2, W_r3])
    B = jnp.stack([b_l1, b_l2, b_l3]).reshape(3, 1, D)
    FL = jnp.array([1.0, 1.0, 0.0], jnp.float32)[:, None, None] \
        * jnp.ones((1, 1, D), jnp.float32)

    def step(h, lyr):
        wl, wr, bb, fl = lyr
        agg, = _agg_k(h, src, ldst)
        h2 = _mm_k(agg.reshape(NC * NH, D), cnt, h, wl, wr, bb, fl)
        return h2, None

    h, _ = lax.scan(step, x, (Wl, Wr, B, FL))
    return h


# speedup vs baseline: 1.5330x; 1.5330x over previous
import jax
import jax.numpy as jnp
from jax import lax
from jax.experimental import pallas as pl
from jax.experimental.pallas import tpu as pltpu
from jax.experimental.pallas import tpu_sc as plsc

N = 10000
D = 128
E = 320000
NC = 2
NS = 16
CH = 128
G = 1
KCH = 158
PAIRS = KCH // 2
GRP = KCH // G
EPAD = NS * KCH * CH
NH = 5120
NHB = NH + 8
RPT = NH // NS
_COPIES = ((0, 128), (128, 128), (256, 64))

_mesh = plsc.VectorSubcoreMesh(core_axis_name="c", subcore_axis_name="s")


def _make_agg():
    scratch = [
        pltpu.VMEM((GRP, G * CH), jnp.int32),
        pltpu.VMEM((GRP, G * CH), jnp.int32),
        pltpu.VMEM((G * CH, D), jnp.float32),
        pltpu.VMEM((G * CH, D), jnp.float32),
        pltpu.VMEM_SHARED((NHB, D), jnp.float32),
        pltpu.SemaphoreType.DMA,
        pltpu.SemaphoreType.DMA,
        pltpu.SemaphoreType.DMA,
    ]

    def body(h_hbm, src_hbm, ldst_hbm, agg_out,
             src_v, ldst_v, rows_v, rows1, agg_sh, gs0, gs1, ss):
        c = lax.axis_index("c")
        s = lax.axis_index("s")

        def zf(i, carry):
            r = i // (D // 16)
            col = (i % (D // 16)) * 16
            rows_v[r, pl.ds(col, 16)] = jnp.zeros((16,), jnp.float32)
            return carry
        lax.fori_loop(0, 128 * (D // 16), zf, 0)

        for off, nr in _COPIES:
            pltpu.sync_copy(rows_v.at[pl.ds(0, nr)],
                            agg_sh.at[pl.ds(s * RPT + off, nr)])

        @pl.when(s == 0)
        def _zero_trash():
            pltpu.sync_copy(rows_v.at[pl.ds(0, 8)], agg_sh.at[pl.ds(NH, 8)])

        pltpu.sync_copy(src_hbm.at[s], src_v)
        pltpu.sync_copy(ldst_hbm.at[c, s], ldst_v)
        plsc.subcore_barrier()

        pltpu.async_copy(h_hbm.at[src_v.at[0]], rows_v, gs0)

        def pair(p, carry):
            j0 = 2 * p
            j1 = j0 + 1
            jn = jnp.minimum(j0 + 2, KCH - 2)
            pltpu.make_async_copy(h_hbm.at[src_v.at[j0]], rows_v, gs0).wait()
            pltpu.async_copy(rows_v, agg_sh.at[ldst_v.at[j0]], ss, add=True)
            pltpu.async_copy(h_hbm.at[src_v.at[j1]], rows1, gs1)
            pltpu.make_async_copy(rows_v, agg_sh.at[ldst_v.at[j0]],
                                  ss).wait()
            pltpu.make_async_copy(h_hbm.at[src_v.at[j1]], rows1, gs1).wait()
            pltpu.async_copy(rows1, agg_sh.at[ldst_v.at[j1]], ss, add=True)
            pltpu.async_copy(h_hbm.at[src_v.at[jn]], rows_v, gs0)
            pltpu.make_async_copy(rows1, agg_sh.at[ldst_v.at[j1]],
                                  ss).wait()
            return carry
        lax.fori_loop(0, PAIRS, pair, 0)
        pltpu.make_async_copy(h_hbm.at[src_v.at[KCH - 2]], rows_v,
                              gs0).wait()
        plsc.subcore_barrier()

        for off, nr in _COPIES:
            o = s * RPT + off
            pltpu.sync_copy(agg_sh.at[pl.ds(o, nr)],
                            agg_out.at[c, pl.ds(o, nr)])

    return pl.kernel(
        body,
        out_type=[jax.ShapeDtypeStruct((NC, NH, D), jnp.float32)],
        mesh=_mesh, scratch_types=scratch)


def _make_cnt():
    scratch = [
        pltpu.VMEM((GRP, G * CH), jnp.int32),
        pltpu.VMEM((G * CH, D), jnp.float32),
        pltpu.VMEM_SHARED((NHB, D), jnp.float32),
        pltpu.SemaphoreType.DMA,
    ]

    def body(ldst_hbm, cnt_out, ldst_v, ones_v, cnt_sh, sem):
        c = lax.axis_index("c")
        s = lax.axis_index("s")

        def zf(i, carry):
            r = i // (D // 16)
            col = (i % (D // 16)) * 16
            ones_v[r, pl.ds(col, 16)] = jnp.zeros((16,), jnp.float32)
            return carry
        lax.fori_loop(0, 128 * (D // 16), zf, 0)

        for off, nr in _COPIES:
            pltpu.sync_copy(ones_v.at[pl.ds(0, nr)],
                            cnt_sh.at[pl.ds(s * RPT + off, nr)])

        @pl.when(s == 0)
        def _zero_trash():
            pltpu.sync_copy(ones_v.at[pl.ds(0, 8)], cnt_sh.at[pl.ds(NH, 8)])

        def of(i, carry):
            r = i // (D // 16)
            col = (i % (D // 16)) * 16
            ones_v[r, pl.ds(col, 16)] = jnp.ones((16,), jnp.float32)
            return carry
        lax.fori_loop(0, G * CH * (D // 16), of, 0)

        pltpu.sync_copy(ldst_hbm.at[c, s], ldst_v)
        plsc.subcore_barrier()

        def step(g, carry):
            pltpu.sync_copy(ones_v, cnt_sh.at[ldst_v.at[g]], add=True)
            return carry
        lax.fori_loop(0, GRP, step, 0)
        plsc.subcore_barrier()

        for off, nr in _COPIES:
            o = s * RPT + off
            pltpu.sync_copy(cnt_sh.at[pl.ds(o, nr)],
                            cnt_out.at[c, pl.ds(o, nr)])

    return pl.kernel(
        body,
        out_type=[jax.ShapeDtypeStruct((NC, NH, D), jnp.float32)],
        mesh=_mesh, scratch_types=scratch)


_agg_k = _make_agg()
_cnt_k = _make_cnt()

R = 1000


def _mm_body(a_ref, cnt_ref, h_ref, wl_ref, wr_ref, b_ref, fl_ref, o_ref):
    inv = 1.0 / jnp.maximum(cnt_ref[:, 0:1], 1.0)
    mean = a_ref[...] * inv
    acc = jnp.dot(mean, wl_ref[...], preferred_element_type=jnp.float32)
    acc = acc + jnp.dot(h_ref[...], wr_ref[...],
                        preferred_element_type=jnp.float32)
    acc = acc + b_ref[...]
    o_ref[...] = jnp.where(fl_ref[...] > 0.0, jnp.maximum(acc, 0.0), acc)


_mm_k = pl.pallas_call(
    _mm_body,
    grid=(N // R,),
    in_specs=[
        pl.BlockSpec((R, D), lambda i: (i, 0)),
        pl.BlockSpec((R, D), lambda i: (i, 0)),
        pl.BlockSpec((R, D), lambda i: (i, 0)),
        pl.BlockSpec((D, D), lambda i: (0, 0)),
        pl.BlockSpec((D, D), lambda i: (0, 0)),
        pl.BlockSpec((1, D), lambda i: (0, 0)),
        pl.BlockSpec((1, D), lambda i: (0, 0)),
    ],
    out_specs=pl.BlockSpec((R, D), lambda i: (i, 0)),
    out_shape=jax.ShapeDtypeStruct((N, D), jnp.float32),
)


@jax.jit
def kernel(x, edge_index, W_l1, b_l1, W_r1, W_l2, b_l2, W_r2,
           W_l3, b_l3, W_r3):
    pad = EPAD - E
    src = jnp.concatenate([edge_index[0],
                           jnp.zeros((pad,), jnp.int32)]).reshape(NS, GRP, G * CH)
    dst = jnp.concatenate([edge_index[1], jnp.full((pad,), -1, jnp.int32)])
    base = jnp.arange(NC, dtype=jnp.int32)[:, None] * NH
    local = dst[None, :] - base
    ldst = jnp.where((local >= 0) & (local < NH), local, NH)
    ldst = ldst.reshape(NC, NS, GRP, G * CH)

    cnt, = _cnt_k(ldst)
    cnt = cnt.reshape(NC * NH, D)

    Wl = jnp.stack([W_l1, W_l2, W_l3])
    Wr = jnp.stack([W_r1, W_r2, W_r3])
    B = jnp.stack([b_l1, b_l2, b_l3]).reshape(3, 1, D)
    FL = jnp.array([1.0, 1.0, 0.0], jnp.float32)[:, None, None] \
        * jnp.ones((1, 1, D), jnp.float32)

    def step(h, lyr):
        wl, wr, bb, fl = lyr
        agg, = _agg_k(h, src, ldst)
        h2 = _mm_k(agg.reshape(NC * NH, D), cnt, h, wl, wr, bb, fl)
        return h2, None

    h, _ = lax.scan(step, x, (Wl, Wr, B, FL))
    return h


# speedup vs baseline: 2.3821x; 1.5539x over previous
import jax
import jax.numpy as jnp
from jax import lax
from jax.experimental import pallas as pl
from jax.experimental.pallas import tpu as pltpu
from jax.experimental.pallas import tpu_sc as plsc

N = 10000
D = 128
E = 320000
NC = 2
NS = 16
CH = 128
KCH = 79
EPAD = NC * NS * KCH * CH
NACC = 10240
NHB = NACC + 8
RPT = NACC // NS

_mesh = plsc.VectorSubcoreMesh(core_axis_name="c", subcore_axis_name="s")


def _make_agg():
    scratch = [
        pltpu.VMEM((KCH, CH), jnp.int32),
        pltpu.VMEM((KCH, CH), jnp.int32),
        pltpu.VMEM((CH, D), jnp.float32),
        pltpu.VMEM_SHARED((NHB, D), jnp.float32),
        pltpu.SemaphoreType.DMA,
    ]

    def body(h_hbm, src_hbm, dst_hbm, agg_out,
             src_v, dst_v, rows_v, agg_sh, sem):
        c = lax.axis_index("c")
        s = lax.axis_index("s")

        def zf(i, carry):
            r = i // (D // 16)
            col = (i % (D // 16)) * 16
            rows_v[r, pl.ds(col, 16)] = jnp.zeros((16,), jnp.float32)
            return carry
        lax.fori_loop(0, CH * (D // 16), zf, 0)

        for off in range(0, RPT, CH):
            pltpu.sync_copy(rows_v, agg_sh.at[pl.ds(s * RPT + off, CH)])

        @pl.when(s == 0)
        def _zero_trash():
            pltpu.sync_copy(rows_v.at[pl.ds(0, 8)], agg_sh.at[pl.ds(NACC, 8)])

        pltpu.sync_copy(src_hbm.at[c, s], src_v)
        pltpu.sync_copy(dst_hbm.at[c, s], dst_v)
        plsc.subcore_barrier()

        def step(g, carry):
            pltpu.sync_copy(h_hbm.at[src_v.at[g]], rows_v)
            pltpu.sync_copy(rows_v, agg_sh.at[dst_v.at[g]], add=True)
            return carry
        lax.fori_loop(0, KCH, step, 0)
        plsc.subcore_barrier()

        pltpu.sync_copy(agg_sh.at[pl.ds(s * RPT, RPT)],
                        agg_out.at[c, pl.ds(s * RPT, RPT)])

    return pl.kernel(
        body,
        out_type=[jax.ShapeDtypeStruct((NC, NACC, D), jnp.float32)],
        mesh=_mesh, scratch_types=scratch)


def _make_cnt():
    scratch = [
        pltpu.VMEM((KCH, CH), jnp.int32),
        pltpu.VMEM((CH, D), jnp.float32),
        pltpu.VMEM_SHARED((NHB, D), jnp.float32),
        pltpu.SemaphoreType.DMA,
    ]

    def body(dst_hbm, cnt_out, dst_v, ones_v, cnt_sh, sem):
        c = lax.axis_index("c")
        s = lax.axis_index("s")

        def zf(i, carry):
            r = i // (D // 16)
            col = (i % (D // 16)) * 16
            ones_v[r, pl.ds(col, 16)] = jnp.zeros((16,), jnp.float32)
            return carry
        lax.fori_loop(0, CH * (D // 16), zf, 0)

        for off in range(0, RPT, CH):
            pltpu.sync_copy(ones_v, cnt_sh.at[pl.ds(s * RPT + off, CH)])

        @pl.when(s == 0)
        def _zero_trash():
            pltpu.sync_copy(ones_v.at[pl.ds(0, 8)], cnt_sh.at[pl.ds(NACC, 8)])

        def of(i, carry):
            r = i // (D // 16)
            col = (i % (D // 16)) * 16
            ones_v[r, pl.ds(col, 16)] = jnp.ones((16,), jnp.float32)
            return carry
        lax.fori_loop(0, CH * (D // 16), of, 0)

        pltpu.sync_copy(dst_hbm.at[c, s], dst_v)
        plsc.subcore_barrier()

        def step(g, carry):
            pltpu.sync_copy(ones_v, cnt_sh.at[dst_v.at[g]], add=True)
            return carry
        lax.fori_loop(0, KCH, step, 0)
        plsc.subcore_barrier()

        pltpu.sync_copy(cnt_sh.at[pl.ds(s * RPT, RPT)],
                        cnt_out.at[c, pl.ds(s * RPT, RPT)])

    return pl.kernel(
        body,
        out_type=[jax.ShapeDtypeStruct((NC, NACC, D), jnp.float32)],
        mesh=_mesh, scratch_types=scratch)


_agg_k = _make_agg()
_cnt_k = _make_cnt()

R = 1000


def _mm_body(a0_ref, a1_ref, cnt_ref, h_ref, wl_ref, wr_ref, b_ref, fl_ref,
             o_ref):
    agg = a0_ref[...] + a1_ref[...]
    inv = 1.0 / jnp.maximum(cnt_ref[:, 0:1], 1.0)
    mean = agg * inv
    acc = jnp.dot(mean, wl_ref[...], preferred_element_type=jnp.float32)
    acc = acc + jnp.dot(h_ref[...], wr_ref[...],
                        preferred_element_type=jnp.float32)
    acc = acc + b_ref[...]
    o_ref[...] = jnp.where(fl_ref[...] > 0.0, jnp.maximum(acc, 0.0), acc)


_mm_k = pl.pallas_call(
    _mm_body,
    grid=(N // R,),
    in_specs=[
        pl.BlockSpec((R, D), lambda i: (i, 0)),
        pl.BlockSpec((R, D), lambda i: (i, 0)),
        pl.BlockSpec((R, D), lambda i: (i, 0)),
        pl.BlockSpec((R, D), lambda i: (i, 0)),
        pl.BlockSpec((D, D), lambda i: (0, 0)),
        pl.BlockSpec((D, D), lambda i: (0, 0)),
        pl.BlockSpec((1, D), lambda i: (0, 0)),
        pl.BlockSpec((1, D), lambda i: (0, 0)),
    ],
    out_specs=pl.BlockSpec((R, D), lambda i: (i, 0)),
    out_shape=jax.ShapeDtypeStruct((N, D), jnp.float32),
)


@jax.jit
def kernel(x, edge_index, W_l1, b_l1, W_r1, W_l2, b_l2, W_r2,
           W_l3, b_l3, W_r3):
    pad = EPAD - E
    src = jnp.concatenate([edge_index[0], jnp.zeros((pad,), jnp.int32)])
    src = src.reshape(NC, NS, KCH, CH)
    dst = jnp.concatenate([edge_index[1],
                           jnp.full((pad,), NACC, jnp.int32)])
    dst = dst.reshape(NC, NS, KCH, CH)

    cntp, = _cnt_k(dst)
    cnt = cntp[0] + cntp[1]

    Wl = jnp.stack([W_l1, W_l2, W_l3])
    Wr = jnp.stack([W_r1, W_r2, W_r3])
    B = jnp.stack([b_l1, b_l2, b_l3]).reshape(3, 1, D)
    FL = jnp.array([1.0, 1.0, 0.0], jnp.float32)[:, None, None] \
        * jnp.ones((1, 1, D), jnp.float32)

    def step(h, lyr):
        wl, wr, bb, fl = lyr
        agg, = _agg_k(h, src, dst)
        h2 = _mm_k(agg[0], agg[1], cnt, h, wl, wr, bb, fl)
        return h2, None

    h, _ = lax.scan(step, x, (Wl, Wr, B, FL))
    return h
